# baseline (speedup 1.0000x reference)
import functools

import jax
import jax.numpy as jnp
from jax.experimental import pallas as pl
from jax.experimental.pallas import tpu as pltpu


def _chamfer_kernel(a_ref, bt_ref, loss_ref, colmin_ref, *, nc, inv_bn, inv_bm):
    c = pl.program_id(1)
    first = (pl.program_id(0) == 0) & (c == 0)

    @pl.when(first)
    def _init():
        loss_ref[...] = jnp.zeros((1, 1), jnp.float32)

    a = a_ref[0]
    bt = bt_ref[0]

    d = jax.lax.dot_general(
        a, bt, (((1,), (0,)), ((), ())),
        precision=jax.lax.Precision.HIGHEST,
        preferred_element_type=jnp.float32,
    )

    rowmin = jnp.min(d, axis=1, keepdims=True)
    loss_ref[...] += jnp.sum(rowmin, keepdims=True) * inv_bn

    cmin = jnp.min(d, axis=0, keepdims=True)

    @pl.when(c == 0)
    def _reset():
        colmin_ref[...] = cmin

    @pl.when(c > 0)
    def _fold():
        colmin_ref[...] = jnp.minimum(colmin_ref[...], cmin)

    @pl.when(c == nc - 1)
    def _finish():
        loss_ref[...] += jnp.sum(colmin_ref[...], keepdims=True) * inv_bm


@jax.jit
def kernel(prediction, target):
    B, N, _ = prediction.shape
    M = target.shape[1]
    TN = 512
    nc = N // TN

    xn = jnp.sum(prediction * prediction, axis=-1, keepdims=True)
    yn = jnp.sum(target * target, axis=-1, keepdims=True)
    ones_x = jnp.ones_like(xn)
    zeros_x = jnp.zeros((B, N, 3), jnp.float32)
    a = jnp.concatenate([-2.0 * prediction, ones_x, xn, zeros_x], axis=-1)
    bt = jnp.transpose(
        jnp.concatenate([target, yn, jnp.ones_like(yn), jnp.zeros((B, M, 3), jnp.float32)], axis=-1),
        (0, 2, 1),
    )

    body = functools.partial(
        _chamfer_kernel,
        nc=nc,
        inv_bn=1.0 / (B * N),
        inv_bm=1.0 / (B * M),
    )
    out = pl.pallas_call(
        body,
        grid=(B, nc),
        in_specs=[
            pl.BlockSpec((1, TN, 8), lambda b, c: (b, c, 0)),
            pl.BlockSpec((1, 8, M), lambda b, c: (b, 0, 0)),
        ],
        out_specs=pl.BlockSpec((1, 1), lambda b, c: (0, 0)),
        out_shape=jax.ShapeDtypeStruct((1, 1), jnp.float32),
        scratch_shapes=[pltpu.VMEM((1, M), jnp.float32)],
    )(a, bt)
    return out[0, 0]
